# baseline (speedup 1.0000x reference)
import functools

import jax
import jax.numpy as jnp
from jax import lax
from jax.experimental import pallas as pl
from jax.experimental.pallas import tpu as pltpu
from jax.experimental.pallas import tpu_sc as plsc

_EPS = 1e-6
_SIGMA_BOOST = 2.0
_B = 64
_K = 16384
_IN = 4096
_OUT = 4096
_NC, _NS, _L = 2, 16, 16
_NW = _NC * _NS

_BH = 32
_NH = _B // _BH
_NR = _BH * _K // 128
_RB = 1024


def _hyper_body(r0_ref, r1_ref, r2_ref, r3_ref, pk0, w0, w1, w2, w3):
    m0 = jax.nn.sigmoid(r0_ref[0]) * (_OUT - 1.0)
    m1 = jax.nn.sigmoid(r1_ref[0]) * (_IN - 1.0)
    sg = jax.nn.softplus(r2_ref[0] + _SIGMA_BOOST) + _EPS
    v = r3_ref[0]
    m0, m1, sg, v = (a.reshape(_RB, 128) for a in (m0, m1, sg, v))
    inv = 1.0 / (sg * float(_OUT) + _EPS)

    f0 = jnp.floor(m0)
    f1 = jnp.floor(m1)
    p0a = f0
    p0b = jnp.minimum(f0 + 1.0, _OUT - 1.0)
    p1a = f1
    p1b = jnp.minimum(f1 + 1.0, _IN - 1.0)

    q0a = (p0a - m0) * (p0a - m0)
    q0b = (p0b - m0) * (p0b - m0)
    q1a = (p1a - m1) * (p1a - m1)
    q1b = (p1b - m1) * (p1b - m1)

    e00 = jnp.exp(-0.5 * (q0a + q1a) * inv)
    e01 = jnp.exp(-0.5 * (q0a + q1b) * inv)
    e10 = jnp.exp(-0.5 * (q0b + q1a) * inv)
    e11 = jnp.exp(-0.5 * (q0b + q1b) * inv)
    scale = v / (e00 + e01 + e10 + e11 + _EPS)

    i0a = p0a.astype(jnp.int32) * _IN
    i1a = p1a.astype(jnp.int32)

    pk0[...] = i0a + i1a
    w0[...] = e00 * scale
    w1[...] = e01 * scale
    w2[...] = e10 * scale
    w3[...] = e11 * scale


_BB = _RB * 128 // _K


def _in_spec(c, h):
    boff = h * _BH // _BB
    return pl.BlockSpec((1, _BB, _K), lambda j, c=c, boff=boff: (c, boff + j, 0))


def _hyper_tc(rt3, h, interpret=False):
    ospec = pl.BlockSpec((_RB, 128), lambda j: (j, 0))
    oshape = jax.ShapeDtypeStruct((_NR, 128), jnp.int32)
    wshape = jax.ShapeDtypeStruct((_NR, 128), jnp.float32)
    return pl.pallas_call(
        _hyper_body,
        grid=(_NR // _RB,),
        in_specs=[_in_spec(c, h) for c in range(4)],
        out_specs=[ospec] * 5,
        out_shape=[oshape] + [wshape] * 4,
        interpret=interpret,
    )(rt3, rt3, rt3, rt3)


_CH = 2048
_NCHUNK = _K // _CH


_UNROLL = 8


def _scatter_body(
    h,
    x_hbm,
    pk_hbm,
    w_hbm0,
    w_hbm1,
    w_hbm2,
    w_hbm3,
    y_hbm,
    acc_v,
    x_v,
    pk_v0,
    wa_v0,
    wb_v0,
    wc_v0,
    wd_v0,
    pk_v1,
    wa_v1,
    wb_v1,
    wc_v1,
    wd_v1,
    y_v,
    sem0,
    sem1,
):
    w_planes = (w_hbm0, w_hbm1, w_hbm2, w_hbm3)
    cid = lax.axis_index("c")
    sid = lax.axis_index("s")
    wid = sid * _NC + cid
    lane = lax.iota(jnp.int32, _L)
    bank = lane * _OUT
    zero16 = jnp.zeros((_L,), jnp.float32)
    bufs = (
        (pk_v0, (wa_v0, wb_v0, wc_v0, wd_v0), sem0),
        (pk_v1, (wa_v1, wb_v1, wc_v1, wd_v1), sem1),
    )

    def _start(b, ch, buf):
        pk_v, wv, sem = bufs[buf]
        base = b * _K + ch * _CH
        hs = [pltpu.async_copy(pk_hbm.at[pl.ds(base, _CH)], pk_v, sem)]
        for c in range(4):
            hs.append(pltpu.async_copy(w_planes[c].at[pl.ds(base, _CH)], wv[c], sem))
        return hs

    @plsc.parallel_loop(0, (_OUT * _L) // _L, 1, unroll=16)
    def _zero(i):
        acc_v[pl.ds(i * _L, _L)] = zero16

    for bi in range(_BH // _NW):
        b = wid * (_BH // _NW) + bi
        pltpu.sync_copy(x_hbm.at[pl.ds((h * _BH + b) * _IN, _IN)], x_v)

        pend = {0: _start(b, 0, 0)}
        for ch in range(_NCHUNK):
            buf = ch % 2
            if ch + 1 < _NCHUNK:
                pend[ch + 1] = _start(b, ch + 1, 1 - buf)
            for hcopy in pend.pop(ch):
                hcopy.wait()
            pk_v, wv, _ = bufs[buf]
            wa_v, wb_v, wc_v, wd_v = wv

            @plsc.parallel_loop(0, _CH // _L, 1, unroll=_UNROLL)
            def _accum(i):
                off = i * _L
                pk = pk_v[pl.ds(off, _L)]
                oid0 = jnp.right_shift(pk, 12)
                iid0 = jnp.bitwise_and(pk, _IN - 1)
                iid1 = jnp.minimum(iid0 + 1, _IN - 1)
                oid1 = jnp.minimum(oid0 + 1, _OUT - 1)
                xa = plsc.load_gather(x_v, [iid0])
                xb = plsc.load_gather(x_v, [iid1])
                c0 = wa_v[pl.ds(off, _L)] * xa + wb_v[pl.ds(off, _L)] * xb
                c1 = wc_v[pl.ds(off, _L)] * xa + wd_v[pl.ds(off, _L)] * xb
                plsc.addupdate_scatter(acc_v, [bank + oid0], c0)
                plsc.addupdate_scatter(acc_v, [bank + oid1], c1)

        @plsc.parallel_loop(0, _OUT // _L, 1, unroll=2)
        def _reduce(g):
            s = acc_v[pl.ds(g * _L, _L)]
            acc_v[pl.ds(g * _L, _L)] = zero16
            for l in range(1, _L):
                off = l * _OUT + g * _L
                s = s + acc_v[pl.ds(off, _L)]
                acc_v[pl.ds(off, _L)] = zero16
            y_v[pl.ds(g * _L, _L)] = s
        pltpu.sync_copy(y_v, y_hbm.at[pl.ds(b * _OUT, _OUT)])


@functools.cache
def _scatter_sc(h):
    chunk_bufs = [
        pltpu.VMEM((_CH,), jnp.int32),
        pltpu.VMEM((_CH,), jnp.float32),
        pltpu.VMEM((_CH,), jnp.float32),
        pltpu.VMEM((_CH,), jnp.float32),
        pltpu.VMEM((_CH,), jnp.float32),
    ]
    return pl.kernel(
        functools.partial(_scatter_body, h),
        out_type=jax.ShapeDtypeStruct((_BH * _OUT,), jnp.float32),
        compiler_params=pltpu.CompilerParams(needs_layout_passes=False),
        mesh=plsc.VectorSubcoreMesh(
            core_axis_name="c", subcore_axis_name="s", num_cores=_NC, num_subcores=_NS
        ),
        scratch_types=[
            pltpu.VMEM((_OUT * _L,), jnp.float32),
            pltpu.VMEM((_IN,), jnp.float32),
        ]
        + chunk_bufs
        + chunk_bufs
        + [
            pltpu.VMEM((_OUT,), jnp.float32),
            pltpu.SemaphoreType.DMA,
            pltpu.SemaphoreType.DMA,
        ],
    )


def kernel(x, res):
    xf = x.reshape(-1)
    rts = [
        jnp.transpose(res[h * _BH : (h + 1) * _BH], (2, 0, 1)) for h in range(_NH)
    ]
    ys = []
    for h in range(_NH):
        planes = _hyper_tc(rts[h], 0)
        flat = [p.reshape(-1) for p in planes]
        ys.append(_scatter_sc(h)(xf, *flat))
    return jnp.concatenate(ys).reshape(_B, _OUT)


# speedup vs baseline: 1.0751x; 1.0751x over previous
import functools

import jax
import jax.numpy as jnp
from jax import lax
from jax.experimental import pallas as pl
from jax.experimental.pallas import tpu as pltpu
from jax.experimental.pallas import tpu_sc as plsc

_EPS = 1e-6
_SIGMA_BOOST = 2.0
_B = 64
_K = 16384
_IN = 4096
_OUT = 4096
_NC, _NS, _L = 2, 16, 16
_NW = _NC * _NS

_BH = 32
_NH = _B // _BH
_NR = _BH * _K // 128
_RB = 1024


def _hyper_body(r0_ref, r1_ref, r2_ref, r3_ref, pk0, w0, w1, w2, w3):
    m0 = jax.nn.sigmoid(r0_ref[0]) * (_OUT - 1.0)
    m1 = jax.nn.sigmoid(r1_ref[0]) * (_IN - 1.0)
    sg = jax.nn.softplus(r2_ref[0] + _SIGMA_BOOST) + _EPS
    v = r3_ref[0]
    m0, m1, sg, v = (a.reshape(_RB, 128) for a in (m0, m1, sg, v))
    inv = 1.0 / (sg * float(_OUT) + _EPS)

    f0 = jnp.floor(m0)
    f1 = jnp.floor(m1)
    p0a = f0
    p0b = jnp.minimum(f0 + 1.0, _OUT - 1.0)
    p1a = f1
    p1b = jnp.minimum(f1 + 1.0, _IN - 1.0)

    q0a = (p0a - m0) * (p0a - m0)
    q0b = (p0b - m0) * (p0b - m0)
    q1a = (p1a - m1) * (p1a - m1)
    q1b = (p1b - m1) * (p1b - m1)

    e00 = jnp.exp(-0.5 * (q0a + q1a) * inv)
    e01 = jnp.exp(-0.5 * (q0a + q1b) * inv)
    e10 = jnp.exp(-0.5 * (q0b + q1a) * inv)
    e11 = jnp.exp(-0.5 * (q0b + q1b) * inv)
    scale = v / (e00 + e01 + e10 + e11 + _EPS)

    i0a = p0a.astype(jnp.int32) * _IN
    i1a = p1a.astype(jnp.int32)

    pk0[...] = i0a + i1a
    w0[...] = e00 * scale
    w1[...] = e01 * scale
    w2[...] = e10 * scale
    w3[...] = e11 * scale


_BB = _RB * 128 // _K


def _in_spec(c, h):
    boff = h * _BH // _BB
    return pl.BlockSpec((1, _BB, _K), lambda j, c=c, boff=boff: (c, boff + j, 0))


def _hyper_tc(rt3, h, interpret=False):
    ospec = pl.BlockSpec((_RB, 128), lambda j: (j, 0))
    oshape = jax.ShapeDtypeStruct((_NR, 128), jnp.int32)
    wshape = jax.ShapeDtypeStruct((_NR, 128), jnp.float32)
    return pl.pallas_call(
        _hyper_body,
        grid=(_NR // _RB,),
        in_specs=[_in_spec(c, h) for c in range(4)],
        out_specs=[ospec] * 5,
        out_shape=[oshape] + [wshape] * 4,
        interpret=interpret,
    )(rt3, rt3, rt3, rt3)


_CH = 4096
_NCHUNK = _K // _CH


_UNROLL = 8


def _scatter_body(
    h,
    x_hbm,
    pk_hbm,
    w_hbm0,
    w_hbm1,
    w_hbm2,
    w_hbm3,
    y_hbm,
    acc_v,
    x_v,
    pk_v0,
    wa_v0,
    wb_v0,
    wc_v0,
    wd_v0,
    pk_v1,
    wa_v1,
    wb_v1,
    wc_v1,
    wd_v1,
    y_v,
    sem0,
    sem1,
):
    w_planes = (w_hbm0, w_hbm1, w_hbm2, w_hbm3)
    cid = lax.axis_index("c")
    sid = lax.axis_index("s")
    wid = sid * _NC + cid
    lane = lax.iota(jnp.int32, _L)
    bank = lane * _OUT
    zero16 = jnp.zeros((_L,), jnp.float32)
    bufs = (
        (pk_v0, (wa_v0, wb_v0, wc_v0, wd_v0), sem0),
        (pk_v1, (wa_v1, wb_v1, wc_v1, wd_v1), sem1),
    )

    def _start(b, ch, buf):
        pk_v, wv, sem = bufs[buf]
        base = b * _K + ch * _CH
        hs = [pltpu.async_copy(pk_hbm.at[pl.ds(base, _CH)], pk_v, sem)]
        for c in range(4):
            hs.append(pltpu.async_copy(w_planes[c].at[pl.ds(base, _CH)], wv[c], sem))
        return hs

    @plsc.parallel_loop(0, (_OUT * _L) // _L, 1, unroll=16)
    def _zero(i):
        acc_v[pl.ds(i * _L, _L)] = zero16

    for bi in range(_BH // _NW):
        b = wid * (_BH // _NW) + bi
        pltpu.sync_copy(x_hbm.at[pl.ds((h * _BH + b) * _IN, _IN)], x_v)

        pend = {0: _start(b, 0, 0)}
        for ch in range(_NCHUNK):
            buf = ch % 2
            if ch + 1 < _NCHUNK:
                pend[ch + 1] = _start(b, ch + 1, 1 - buf)
            for hcopy in pend.pop(ch):
                hcopy.wait()
            pk_v, wv, _ = bufs[buf]
            wa_v, wb_v, wc_v, wd_v = wv

            @plsc.parallel_loop(0, _CH // _L, 1, unroll=_UNROLL)
            def _accum(i):
                off = i * _L
                pk = pk_v[pl.ds(off, _L)]
                oid0 = jnp.right_shift(pk, 12)
                iid0 = jnp.bitwise_and(pk, _IN - 1)
                iid1 = jnp.minimum(iid0 + 1, _IN - 1)
                oid1 = jnp.minimum(oid0 + 1, _OUT - 1)
                xa = plsc.load_gather(x_v, [iid0])
                xb = plsc.load_gather(x_v, [iid1])
                c0 = wa_v[pl.ds(off, _L)] * xa + wb_v[pl.ds(off, _L)] * xb
                c1 = wc_v[pl.ds(off, _L)] * xa + wd_v[pl.ds(off, _L)] * xb
                plsc.addupdate_scatter(acc_v, [bank + oid0], c0)
                plsc.addupdate_scatter(acc_v, [bank + oid1], c1)

        @plsc.parallel_loop(0, _OUT // _L, 1, unroll=2)
        def _reduce(g):
            s = acc_v[pl.ds(g * _L, _L)]
            acc_v[pl.ds(g * _L, _L)] = zero16
            for l in range(1, _L):
                off = l * _OUT + g * _L
                s = s + acc_v[pl.ds(off, _L)]
                acc_v[pl.ds(off, _L)] = zero16
            y_v[pl.ds(g * _L, _L)] = s
        pltpu.sync_copy(y_v, y_hbm.at[pl.ds(b * _OUT, _OUT)])


@functools.cache
def _scatter_sc(h):
    chunk_bufs = [
        pltpu.VMEM((_CH,), jnp.int32),
        pltpu.VMEM((_CH,), jnp.float32),
        pltpu.VMEM((_CH,), jnp.float32),
        pltpu.VMEM((_CH,), jnp.float32),
        pltpu.VMEM((_CH,), jnp.float32),
    ]
    return pl.kernel(
        functools.partial(_scatter_body, h),
        out_type=jax.ShapeDtypeStruct((_BH * _OUT,), jnp.float32),
        compiler_params=pltpu.CompilerParams(needs_layout_passes=False),
        mesh=plsc.VectorSubcoreMesh(
            core_axis_name="c", subcore_axis_name="s", num_cores=_NC, num_subcores=_NS
        ),
        scratch_types=[
            pltpu.VMEM((_OUT * _L,), jnp.float32),
            pltpu.VMEM((_IN,), jnp.float32),
        ]
        + chunk_bufs
        + chunk_bufs
        + [
            pltpu.VMEM((_OUT,), jnp.float32),
            pltpu.SemaphoreType.DMA,
            pltpu.SemaphoreType.DMA,
        ],
    )


def kernel(x, res):
    xf = x.reshape(-1)
    rt3 = jnp.transpose(res, (2, 0, 1))
    ys = []
    for h in range(_NH):
        planes = _hyper_tc(rt3, h)
        flat = [p.reshape(-1) for p in planes]
        ys.append(_scatter_sc(h)(xf, *flat))
    return jnp.concatenate(ys).reshape(_B, _OUT)
